# baseline (speedup 1.0000x reference)
import functools

import jax
import jax.numpy as jnp
from jax import lax
from jax.experimental import pallas as pl
from jax.experimental.pallas import tpu as pltpu
from jax.experimental.pallas import tpu_sc as plsc

_D = 64
_STREAM = 128
_CHUNK = 1024
_K = _CHUNK // _STREAM
_NW = 32


@functools.partial(jax.jit, static_argnums=(2,))
def _gather(table, idx2d, n_total):
    per_w = n_total // _NW
    n_chunks = per_w // _CHUNK
    rows_per_w = per_w // _STREAM
    mesh = plsc.VectorSubcoreMesh(core_axis_name="c", subcore_axis_name="s")

    @functools.partial(
        pl.kernel,
        mesh=mesh,
        out_type=jax.ShapeDtypeStruct((n_total, _D), jnp.float32),
        scratch_types=[
            pltpu.VMEM((_K, _STREAM), jnp.int32),
            pltpu.VMEM((_CHUNK, _D), jnp.float32),
            pltpu.SemaphoreType.DMA,
        ],
        compiler_params=pltpu.CompilerParams(use_tc_tiling_on_sc=False),
    )
    def k(table_hbm, idx_hbm, out_hbm, idx_v, rows_v, sem):
        wid = lax.axis_index("s") * 2 + lax.axis_index("c")
        row0 = wid * rows_per_w

        def chunk_body(c, carry):
            r = row0 + c * _K
            pltpu.sync_copy(idx_hbm.at[pl.ds(r, _K)], idx_v)
            copies = [
                pltpu.async_copy(
                    table_hbm.at[idx_v.at[j]],
                    rows_v.at[pl.ds(j * _STREAM, _STREAM)],
                    sem,
                )
                for j in range(_K)
            ]
            for cp in copies:
                cp.wait()
            pltpu.sync_copy(rows_v, out_hbm.at[pl.ds(r * _STREAM, _CHUNK)])
            return carry

        lax.fori_loop(0, n_chunks, chunk_body, 0)

    return k(table, idx2d)


def kernel(x, table):
    n_total = x.size
    idx2d = x.reshape(n_total // _STREAM, _STREAM).astype(jnp.int32)
    out = _gather(table, idx2d, n_total)
    return out.reshape(*x.shape, _D)


# speedup vs baseline: 1.0140x; 1.0140x over previous
import functools

import jax
import jax.numpy as jnp
from jax import lax
from jax.experimental import pallas as pl
from jax.experimental.pallas import tpu as pltpu
from jax.experimental.pallas import tpu_sc as plsc

_D = 64
_STREAM = 128
_CHUNK = 512
_K = _CHUNK // _STREAM
_NW = 32


@functools.partial(jax.jit, static_argnums=(2,))
def _gather(table, idx2d, n_total):
    per_w = n_total // _NW
    rows_per_w = per_w // _STREAM
    n_chunks = per_w // _CHUNK
    n_pairs = n_chunks // 2
    mesh = plsc.VectorSubcoreMesh(core_axis_name="c", subcore_axis_name="s")

    @functools.partial(
        pl.kernel,
        mesh=mesh,
        out_type=jax.ShapeDtypeStruct((n_total, _D), jnp.float32),
        scratch_types=[
            pltpu.VMEM((rows_per_w, _STREAM), jnp.int32),
            pltpu.VMEM((_CHUNK, _D), jnp.float32),
            pltpu.VMEM((_CHUNK, _D), jnp.float32),
            pltpu.SemaphoreType.DMA,
            pltpu.SemaphoreType.DMA,
            pltpu.SemaphoreType.DMA,
            pltpu.SemaphoreType.DMA,
        ],
        compiler_params=pltpu.CompilerParams(use_tc_tiling_on_sc=False),
    )
    def k(table_hbm, idx_hbm, out_hbm, idx_v, rows0, rows1,
          semg0, semg1, semo0, semo1):
        wid = lax.axis_index("s") * 2 + lax.axis_index("c")
        row0 = wid * rows_per_w
        out0 = wid * per_w

        pltpu.sync_copy(idx_hbm.at[pl.ds(row0, rows_per_w)], idx_v)

        rows = (rows0, rows1)
        semg = (semg0, semg1)
        semo = (semo0, semo1)

        def fire_g(c, b):
            for j in range(_K):
                pltpu.make_async_copy(
                    table_hbm.at[idx_v.at[c * _K + j]],
                    rows[b].at[pl.ds(j * _STREAM, _STREAM)],
                    semg[b],
                ).start()

        def drain_g(b):
            for j in range(_K):
                pltpu.make_async_copy(
                    table_hbm.at[pl.ds(0, _STREAM)],
                    rows[b].at[pl.ds(j * _STREAM, _STREAM)],
                    semg[b],
                ).wait()

        def fire_w(c, b):
            pltpu.make_async_copy(
                rows[b], out_hbm.at[pl.ds(out0 + c * _CHUNK, _CHUNK)], semo[b]
            ).start()

        def wait_w(b):
            pltpu.make_async_copy(
                rows[b], out_hbm.at[pl.ds(out0, _CHUNK)], semo[b]
            ).wait()

        fire_g(0, 0)
        drain_g(0)
        fire_w(0, 0)
        fire_g(1, 1)
        drain_g(1)
        fire_w(1, 1)
        wait_w(0)
        fire_g(2, 0)

        def pair_body(p, carry):
            c0 = 2 * p
            drain_g(0)
            fire_w(c0, 0)
            wait_w(1)
            fire_g(c0 + 1, 1)
            drain_g(1)
            fire_w(c0 + 1, 1)
            wait_w(0)
            fire_g(c0 + 2, 0)
            return carry

        lax.fori_loop(1, n_pairs - 1, pair_body, 0)

        c0 = n_chunks - 2
        drain_g(0)
        fire_w(c0, 0)
        wait_w(1)
        fire_g(c0 + 1, 1)
        drain_g(1)
        fire_w(c0 + 1, 1)
        wait_w(0)
        wait_w(1)

    return k(table, idx2d)


def kernel(x, table):
    n_total = x.size
    idx2d = x.reshape(n_total // _STREAM, _STREAM).astype(jnp.int32)
    out = _gather(table, idx2d, n_total)
    return out.reshape(*x.shape, _D)
